# baseline (speedup 1.0000x reference)
import functools

import jax
import jax.numpy as jnp
from jax import lax
from jax.experimental import pallas as pl
from jax.experimental.pallas import tpu as pltpu
from jax.experimental.pallas import tpu_sc as plsc

B = 256
BC = 64
NCHUNK = B // BC
NC, NS = 2, 16
NW = NC * NS
UNROLL = 16


@functools.lru_cache(maxsize=None)
def _make_layer(I, O):
    npw = O // NW
    mesh = plsc.VectorSubcoreMesh(core_axis_name="c", subcore_axis_name="s",
                                  num_cores=NC, num_subcores=NS)

    @functools.partial(
        pl.kernel,
        out_type=jax.ShapeDtypeStruct((NCHUNK, O, BC), jnp.uint16),
        mesh=mesh,
        compiler_params=pltpu.CompilerParams(use_tc_tiling_on_sc=False),
        scratch_types=[
            pltpu.VMEM((2 * I, BC), jnp.uint16),
            pltpu.VMEM((npw, I, 32), jnp.uint16),
            pltpu.VMEM((16,), jnp.int32),
            pltpu.VMEM((npw, BC), jnp.uint16),
        ],
    )
    def layer(z_hbm, m_hbm, base_hbm, out_hbm, z_v, m_v, b_v, o_v):
        c = lax.axis_index("c")
        s = lax.axis_index("s")
        w = s * NC + c
        pltpu.sync_copy(m_hbm.at[pl.ds(w * npw, npw)], m_v)
        pltpu.sync_copy(base_hbm.at[w], b_v)
        bvec = b_v[...]

        def chunk(ci, carry):
            pltpu.sync_copy(z_hbm.at[ci], z_v)
            for o in range(npw):
                base = bvec[o]

                def body(iu, accs, o=o, base=base):
                    a0, a1 = accs
                    i0 = iu * UNROLL
                    for u in range(UNROLL):
                        mm = m_v[o, i0 + u, :]
                        z0 = z_v[base + i0 + u, pl.ds(0, 32)]
                        z1 = z_v[base + i0 + u, pl.ds(32, 32)]
                        a0 = jnp.maximum(a0, z0 & mm)
                        a1 = jnp.maximum(a1, z1 & mm)
                    return a0, a1

                zero = jnp.zeros((32,), jnp.uint16)
                a0, a1 = lax.fori_loop(0, I // UNROLL, body, (zero, zero))
                o_v[o, pl.ds(0, 32)] = a0
                o_v[o, pl.ds(32, 32)] = a1
            pltpu.sync_copy(o_v, out_hbm.at[ci, pl.ds(w * npw, npw), :])
            return carry

        lax.fori_loop(0, NCHUNK, chunk, 0)

    return layer


def _node_params(ntc, g, I):
    logits = ntc + g
    y_soft = jax.nn.softmax(logits, axis=-1)
    amax = jnp.argmax(logits, axis=-1)
    y_hard = jax.nn.one_hot(amax, 2, dtype=logits.dtype)
    sel = y_soft + (y_hard - y_soft)
    is_max = amax == 1
    base = jnp.where(is_max, 0, I).astype(jnp.int32)
    p = jnp.where(is_max, 0.0, sel[:, 0])
    q = jnp.where(is_max, sel[:, 1], -sel[:, 0])
    return base, p, q


def _pack_worker(a, npw, width=16):
    a = a.reshape(NW, npw)
    pad = jnp.zeros((NW, width - npw), a.dtype)
    return jnp.concatenate([a, pad], axis=1)


def _to_z(f):
    u = lax.bitcast_convert_type(f.astype(jnp.bfloat16), jnp.uint16)
    rows = f.shape[0]
    return u.reshape(rows, NCHUNK, BC).transpose(1, 0, 2)


def _post(red_u16, p, q):
    red = lax.bitcast_convert_type(red_u16, jnp.bfloat16).astype(jnp.float32)
    return p[None, :, None] + q[None, :, None] * red


def kernel(x, ntc1, ntc2, g1, g2, noedge1, noedge2):
    b1, p1, q1 = _node_params(ntc1, g1, 512)
    b2, p2, q2 = _node_params(ntc2, g2, 256)
    m1 = jnp.where(noedge1, 0, 0xFFFF).astype(jnp.uint16)
    m2 = jnp.where(noedge2, 0, 0xFFFF).astype(jnp.uint16)
    m1b = jnp.broadcast_to(m1[:, :, None], (256, 512, 32))
    m2b = jnp.broadcast_to(m2[:, :, None], (512, 256, 32))

    z1 = _to_z(jnp.concatenate([x.T, (1.0 - x).T], axis=0))
    r1 = _make_layer(512, 256)(z1, m1b, _pack_worker(b1, 256 // NW))
    h = _post(r1, p1, q1)

    hh = jnp.concatenate([h, 1.0 - h], axis=1).astype(jnp.bfloat16)
    z2 = lax.bitcast_convert_type(hh, jnp.uint16)
    r2 = _make_layer(256, 512)(z2, m2b, _pack_worker(b2, 512 // NW))
    out = _post(r2, p2, q2)
    return out.transpose(0, 2, 1).reshape(B, 512)
